# baseline (speedup 1.0000x reference)
import functools
import math

import jax
import jax.numpy as jnp
from jax import lax
from jax.experimental import pallas as pl
from jax.experimental.pallas import tpu as pltpu
from jax.experimental.pallas import tpu_sc as plsc

D_MODEL = 64
SCALE = math.sqrt(D_MODEL)

NC, NS, L = 2, 16, 16
NW = NC * NS

B_TOTAL = 4096 * 200
CHUNK = 128
CHUNKS_TOTAL = B_TOTAL // CHUNK
CHUNKS_PER_W = CHUNKS_TOTAL // NW
NBUF = 4


def _sc_embed(x2d, lut):
    mesh = plsc.VectorSubcoreMesh(core_axis_name="c", subcore_axis_name="s")

    @functools.partial(
        pl.kernel,
        mesh=mesh,
        out_type=jax.ShapeDtypeStruct((B_TOTAL, D_MODEL), jnp.float32),
        scratch_types=[
            pltpu.VMEM((CHUNKS_PER_W, CHUNK), jnp.int32),
            pltpu.VMEM((NBUF * CHUNK, D_MODEL), jnp.float32),
        ] + [pltpu.SemaphoreType.DMA] * NBUF,
        compiler_params=pltpu.CompilerParams(use_tc_tiling_on_sc=False),
    )
    def k(x_hbm, lut_hbm, out_hbm, idx_v, rows_v, *sems):
        c = lax.axis_index("c")
        s = lax.axis_index("s")
        wid = s * NC + c
        chunk0 = wid * CHUNKS_PER_W

        pltpu.sync_copy(x_hbm.at[pl.ds(chunk0, CHUNKS_PER_W)], idx_v)

        def start_gather(g, b):
            pltpu.async_copy(
                lut_hbm.at[idx_v.at[g]],
                rows_v.at[pl.ds(b * CHUNK, CHUNK)],
                sems[b],
            )

        def wait_gather(g, b):
            pltpu.make_async_copy(
                lut_hbm.at[idx_v.at[g]],
                rows_v.at[pl.ds(b * CHUNK, CHUNK)],
                sems[b],
            ).wait()

        for b in range(NBUF):
            start_gather(b, b)

        def outer(i, carry):
            g0 = i * NBUF
            for b in range(NBUF):
                g = g0 + b
                wait_gather(g, b)

                def scale_row(r, acc, b=b):
                    row = b * CHUNK + r
                    for d in range(D_MODEL // L):
                        sl = (row, pl.ds(d * L, L))
                        rows_v[sl] = rows_v[sl] * SCALE
                    return acc

                lax.fori_loop(0, CHUNK, scale_row, 0, unroll=2)

                pltpu.sync_copy(
                    rows_v.at[pl.ds(b * CHUNK, CHUNK)],
                    out_hbm.at[pl.ds((chunk0 + g) * CHUNK, CHUNK)],
                )

                @pl.when(g + NBUF < CHUNKS_PER_W)
                def _refill(g=g, b=b):
                    start_gather(g + NBUF, b)
            return carry

        lax.fori_loop(0, CHUNKS_PER_W // NBUF, outer, 0)

    return k(x2d, lut)


def kernel(x, lut):
    x2d = x.reshape(CHUNKS_TOTAL, CHUNK)
    out = _sc_embed(x2d, lut)
    return out.reshape(4096, 200, D_MODEL)
